# baseline (speedup 1.0000x reference)
import functools

import jax
import jax.numpy as jnp
from jax import lax
from jax.experimental import pallas as pl
from jax.experimental.pallas import tpu as pltpu
from jax.experimental.pallas import tpu_sc as plsc

_N = 10000
_D = 128
_E = 160000
_NB = 4
_NG = 9
_ACC_N = 10240
_NW = 32
_CHUNK = 128
_E_PAD = 163840
_E_TILE = _E_PAD // _NW
_NCH = _E_TILE // _CHUNK
_ROWS_PER_TILE = _ACC_N // 16
_BN = 1000


def _project(x, w_ref, c_ref, s_ref, y_ref):
    xb = [jnp.dot(x, w_ref[b], preferred_element_type=jnp.float32)
          for b in range(_NB)]
    for r in range(_NG - 1):
        acc = c_ref[r, 0] * xb[0]
        for b in range(1, _NB):
            acc = acc + c_ref[r, b] * xb[b]
        y_ref[r, :, :] = acc
    y_ref[_NG - 1, :, :] = jnp.dot(
        x, s_ref[...], preferred_element_type=jnp.float32)


def _mm1_kernel(x_ref, w_ref, c_ref, s_ref, y_ref):
    _project(x_ref[...], w_ref, c_ref, s_ref, y_ref)


def _mm2_kernel(p0_ref, p1_ref, sp_ref, w_ref, c_ref, s_ref, h_ref, y_ref):
    h = jnp.maximum(p0_ref[0] + p1_ref[0] + sp_ref[0], 0.0)
    h_ref[...] = h
    _project(h, w_ref, c_ref, s_ref, y_ref)


def _relu_kernel(p0_ref, p1_ref, sp_ref, h_ref):
    h_ref[...] = jnp.maximum(p0_ref[0] + p1_ref[0] + sp_ref[0], 0.0)


_GRID = _N // _BN

_W_SPEC = pl.BlockSpec((_NB, _D, _D), lambda i: (0, 0, 0))
_C_SPEC = pl.BlockSpec(memory_space=pltpu.SMEM)
_S_SPEC = pl.BlockSpec((_D, _D), lambda i: (0, 0))
_X_SPEC = pl.BlockSpec((_BN, _D), lambda i: (i, 0))
_Y_SPEC = pl.BlockSpec((_NG, _BN, _D), lambda i: (0, i, 0))
_P0_SPEC = pl.BlockSpec((1, _BN, _D), lambda i: (0, i, 0))
_P1_SPEC = pl.BlockSpec((1, _BN, _D), lambda i: (1, i, 0))
_SP_SPEC = pl.BlockSpec((1, _BN, _D), lambda i: (_NG - 1, i, 0))

_mm1 = pl.pallas_call(
    _mm1_kernel,
    grid=(_GRID,),
    in_specs=[_X_SPEC, _W_SPEC, _C_SPEC, _S_SPEC],
    out_specs=_Y_SPEC,
    out_shape=jax.ShapeDtypeStruct((_NG, _N, _D), jnp.float32),
)

_mm2 = pl.pallas_call(
    _mm2_kernel,
    grid=(_GRID,),
    in_specs=[_P0_SPEC, _P1_SPEC, _SP_SPEC, _W_SPEC, _C_SPEC, _S_SPEC],
    out_specs=[_X_SPEC, _Y_SPEC],
    out_shape=[
        jax.ShapeDtypeStruct((_N, _D), jnp.float32),
        jax.ShapeDtypeStruct((_NG, _N, _D), jnp.float32),
    ],
)

_relu = pl.pallas_call(
    _relu_kernel,
    grid=(_GRID,),
    in_specs=[_P0_SPEC, _P1_SPEC, _SP_SPEC],
    out_specs=_X_SPEC,
    out_shape=jax.ShapeDtypeStruct((_N, _D), jnp.float32),
)


_DEPTH = 2
_Q0 = 16
_Q1 = (_NW * _NCH - 16 * _Q0) // 16
_QMAX = max(_Q0, _Q1)


@functools.cache
def _make_sc_scatter():
    mesh = plsc.VectorSubcoreMesh(core_axis_name="c", subcore_axis_name="s")

    @functools.partial(
        pl.kernel,
        out_type=jax.ShapeDtypeStruct((2, _ACC_N, _D), jnp.float32),
        mesh=mesh,
        scratch_types=(
            [pltpu.VMEM((_QMAX, _CHUNK), jnp.int32)] * 2
            + [pltpu.VMEM((_CHUNK, _D), jnp.float32)] * _DEPTH
            + [pltpu.VMEM_SHARED((_ACC_N, _D), jnp.float32)]
            + [pltpu.SemaphoreType.DMA] * (2 * _DEPTH)
        ),
    )
    def _sc_scatter(table_hbm, gidx_hbm, sidx_hbm, zeros_hbm, out_hbm,
                    gi_v, si_v, *scr):
        rows = scr[:_DEPTH]
        acc_sh = scr[_DEPTH]
        gsem = scr[_DEPTH + 1:2 * _DEPTH + 1]
        ssem = scr[2 * _DEPTH + 1:]
        cid = lax.axis_index("c")
        sid = lax.axis_index("s")
        ngen = jnp.where(cid == 0, _Q0 // _DEPTH, _Q1 // _DEPTH)

        def wait_gather(b):
            pltpu.make_async_copy(
                table_hbm.at[gi_v.at[b]], rows[b], gsem[b]).wait()

        def wait_scatter(b):
            pltpu.make_async_copy(
                rows[b], acc_sh.at[si_v.at[b]], ssem[b]).wait()

        @pl.when(cid == 0)
        def _():
            cbase = sid * _Q0
            pltpu.sync_copy(gidx_hbm.at[pl.ds(cbase, _Q0)],
                            gi_v.at[pl.ds(0, _Q0)])
            pltpu.sync_copy(sidx_hbm.at[pl.ds(cbase, _Q0)],
                            si_v.at[pl.ds(0, _Q0)])

        @pl.when(cid == 1)
        def _():
            cbase = 16 * _Q0 + sid * _Q1
            pltpu.sync_copy(gidx_hbm.at[pl.ds(cbase, _Q1)],
                            gi_v.at[pl.ds(0, _Q1)])
            pltpu.sync_copy(sidx_hbm.at[pl.ds(cbase, _Q1)],
                            si_v.at[pl.ds(0, _Q1)])

        for b in range(_DEPTH):
            pltpu.async_copy(table_hbm.at[gi_v.at[b]], rows[b], gsem[b])
        row0 = pl.multiple_of(sid * _ROWS_PER_TILE, _ROWS_PER_TILE)
        pltpu.sync_copy(zeros_hbm, acc_sh.at[pl.ds(row0, _ROWS_PER_TILE)])
        plsc.subcore_barrier()

        def gen_body(g, carry):
            base_j = g * _DEPTH
            for b in range(_DEPTH):
                wait_gather(b)
                pltpu.async_copy(
                    rows[b], acc_sh.at[si_v.at[base_j + b]], ssem[b],
                    add=True)
            for b in range(_DEPTH):
                wait_scatter(b)

                @pl.when(g + 1 < ngen)
                def _():
                    jn = base_j + _DEPTH + b
                    pltpu.async_copy(
                        table_hbm.at[gi_v.at[jn]], rows[b], gsem[b])
            return carry

        lax.fori_loop(0, ngen, gen_body, 0)
        plsc.subcore_barrier()
        pltpu.sync_copy(acc_sh.at[pl.ds(row0, _ROWS_PER_TILE)],
                        out_hbm.at[cid, pl.ds(row0, _ROWS_PER_TILE)])

    return _sc_scatter


def kernel(x, edge_index, edge_type, weight1, w_comp1, self_loop1,
           weight2, w_comp2, self_loop2):
    src = edge_index[0]
    dst = edge_index[1]
    pad = _E_PAD - _E
    gidx = jnp.concatenate(
        [edge_type * _N + src, jnp.zeros((pad,), jnp.int32)])
    sidx = jnp.concatenate(
        [dst, _N + (jnp.arange(pad, dtype=jnp.int32) % (_ACC_N - _N))])
    gidx = gidx.reshape(_NW * _NCH, _CHUNK)
    sidx = sidx.reshape(_NW * _NCH, _CHUNK)
    zeros = jnp.zeros((_ROWS_PER_TILE, _D), jnp.float32)

    sc_scatter = _make_sc_scatter()
    y1 = _mm1(x, weight1, w_comp1, self_loop1)
    parts1 = sc_scatter(y1.reshape(_NG * _N, _D), gidx, sidx, zeros)
    h1, y2 = _mm2(parts1, parts1, y1, weight2, w_comp2, self_loop2)
    parts2 = sc_scatter(y2.reshape(_NG * _N, _D), gidx, sidx, zeros)
    h2 = _relu(parts2, parts2, y2)
    return jnp.stack([h1, h2], axis=1)


# speedup vs baseline: 1.2186x; 1.2186x over previous
import functools

import jax
import jax.numpy as jnp
from jax import lax
from jax.experimental import pallas as pl
from jax.experimental.pallas import tpu as pltpu
from jax.experimental.pallas import tpu_sc as plsc

_N = 10000
_D = 128
_E = 160000
_NB = 4
_NG = 9
_ACC_N = 10240
_NW = 32
_CHUNK = 128
_E_PAD = 163840
_E_TILE = _E_PAD // _NW
_NCH = _E_TILE // _CHUNK
_ROWS_PER_TILE = _ACC_N // 16
_BN = 1000


def _project(x, w_ref, c_ref, s_ref, y_ref):
    xb = [jnp.dot(x, w_ref[b], preferred_element_type=jnp.float32)
          for b in range(_NB)]
    for r in range(_NG - 1):
        acc = c_ref[r, 0] * xb[0]
        for b in range(1, _NB):
            acc = acc + c_ref[r, b] * xb[b]
        y_ref[r, :, :] = acc
    y_ref[_NG - 1, :, :] = jnp.dot(
        x, s_ref[...], preferred_element_type=jnp.float32)


def _mm1_kernel(x_ref, w_ref, c_ref, s_ref, y_ref):
    _project(x_ref[...], w_ref, c_ref, s_ref, y_ref)


def _mm2_kernel(p0_ref, p1_ref, sp_ref, w_ref, c_ref, s_ref, h_ref, y_ref):
    h = jnp.maximum(p0_ref[0] + p1_ref[0] + sp_ref[0], 0.0)
    h_ref[...] = h
    _project(h, w_ref, c_ref, s_ref, y_ref)


def _relu_kernel(p0_ref, p1_ref, sp_ref, h_ref):
    h_ref[...] = jnp.maximum(p0_ref[0] + p1_ref[0] + sp_ref[0], 0.0)


_GRID = _N // _BN

_W_SPEC = pl.BlockSpec((_NB, _D, _D), lambda i: (0, 0, 0))
_C_SPEC = pl.BlockSpec(memory_space=pltpu.SMEM)
_S_SPEC = pl.BlockSpec((_D, _D), lambda i: (0, 0))
_X_SPEC = pl.BlockSpec((_BN, _D), lambda i: (i, 0))
_Y_SPEC = pl.BlockSpec((_NG, _BN, _D), lambda i: (0, i, 0))
_P0_SPEC = pl.BlockSpec((1, _BN, _D), lambda i: (0, i, 0))
_P1_SPEC = pl.BlockSpec((1, _BN, _D), lambda i: (1, i, 0))
_SP_SPEC = pl.BlockSpec((1, _BN, _D), lambda i: (_NG - 1, i, 0))

_mm1 = pl.pallas_call(
    _mm1_kernel,
    grid=(_GRID,),
    in_specs=[_X_SPEC, _W_SPEC, _C_SPEC, _S_SPEC],
    out_specs=_Y_SPEC,
    out_shape=jax.ShapeDtypeStruct((_NG, _N, _D), jnp.float32),
)

_mm2 = pl.pallas_call(
    _mm2_kernel,
    grid=(_GRID,),
    in_specs=[_P0_SPEC, _P1_SPEC, _SP_SPEC, _W_SPEC, _C_SPEC, _S_SPEC],
    out_specs=[_X_SPEC, _Y_SPEC],
    out_shape=[
        jax.ShapeDtypeStruct((_N, _D), jnp.float32),
        jax.ShapeDtypeStruct((_NG, _N, _D), jnp.float32),
    ],
)

_relu = pl.pallas_call(
    _relu_kernel,
    grid=(_GRID,),
    in_specs=[_P0_SPEC, _P1_SPEC, _SP_SPEC],
    out_specs=_X_SPEC,
    out_shape=jax.ShapeDtypeStruct((_N, _D), jnp.float32),
)


_DEPTH = 2
_Q0 = 40
_Q1 = (_NW * _NCH - 16 * _Q0) // 16
_QMAX = max(_Q0, _Q1)


@functools.cache
def _make_sc_scatter():
    mesh = plsc.VectorSubcoreMesh(core_axis_name="c", subcore_axis_name="s")

    @functools.partial(
        pl.kernel,
        out_type=jax.ShapeDtypeStruct((2, _ACC_N, _D), jnp.float32),
        mesh=mesh,
        scratch_types=(
            [pltpu.VMEM((_QMAX, _CHUNK), jnp.int32)] * 2
            + [pltpu.VMEM((_CHUNK, _D), jnp.float32)] * _DEPTH
            + [pltpu.VMEM_SHARED((_ACC_N, _D), jnp.float32)]
            + [pltpu.SemaphoreType.DMA] * (2 * _DEPTH)
        ),
    )
    def _sc_scatter(table_hbm, gidx_hbm, sidx_hbm, out_hbm,
                    gi_v, si_v, *scr):
        rows = scr[:_DEPTH]
        acc_sh = scr[_DEPTH]
        gsem = scr[_DEPTH + 1:2 * _DEPTH + 1]
        ssem = scr[2 * _DEPTH + 1:]
        cid = lax.axis_index("c")
        sid = lax.axis_index("s")
        ngen = jnp.where(cid == 0, _Q0 // _DEPTH, _Q1 // _DEPTH)

        def wait_gather(b):
            pltpu.make_async_copy(
                table_hbm.at[gi_v.at[b]], rows[b], gsem[b]).wait()

        def wait_scatter(b):
            pltpu.make_async_copy(
                rows[b], acc_sh.at[si_v.at[b]], ssem[b]).wait()

        @pl.when(cid == 0)
        def _():
            cbase = sid * _Q0
            pltpu.sync_copy(gidx_hbm.at[pl.ds(cbase, _Q0)],
                            gi_v.at[pl.ds(0, _Q0)])
            pltpu.sync_copy(sidx_hbm.at[pl.ds(cbase, _Q0)],
                            si_v.at[pl.ds(0, _Q0)])

        @pl.when(cid == 1)
        def _():
            cbase = 16 * _Q0 + sid * _Q1
            pltpu.sync_copy(gidx_hbm.at[pl.ds(cbase, _Q1)],
                            gi_v.at[pl.ds(0, _Q1)])
            pltpu.sync_copy(sidx_hbm.at[pl.ds(cbase, _Q1)],
                            si_v.at[pl.ds(0, _Q1)])

        zv = jnp.zeros((16,), jnp.float32)

        def zrow(i, carry):
            for j in range(8):
                rows[0][i, 16 * j:16 * (j + 1)] = zv
            return carry

        lax.fori_loop(0, _CHUNK, zrow, 0)
        row0 = pl.multiple_of(sid * _ROWS_PER_TILE, _ROWS_PER_TILE)
        for z in range(_ROWS_PER_TILE // _CHUNK):
            pltpu.async_copy(
                rows[0], acc_sh.at[pl.ds(row0 + z * _CHUNK, _CHUNK)],
                ssem[0])
        for z in range(_ROWS_PER_TILE // _CHUNK):
            pltpu.make_async_copy(
                rows[0], acc_sh.at[pl.ds(row0, _CHUNK)], ssem[0]).wait()
        for b in range(_DEPTH):
            pltpu.async_copy(table_hbm.at[gi_v.at[b]], rows[b], gsem[b])
        plsc.subcore_barrier()

        def gen_body(g, carry):
            base_j = g * _DEPTH
            for b in range(_DEPTH):
                wait_gather(b)
                pltpu.async_copy(
                    rows[b], acc_sh.at[si_v.at[base_j + b]], ssem[b],
                    add=True)
            for b in range(_DEPTH):
                wait_scatter(b)

                @pl.when(g + 1 < ngen)
                def _():
                    jn = base_j + _DEPTH + b
                    pltpu.async_copy(
                        table_hbm.at[gi_v.at[jn]], rows[b], gsem[b])
            return carry

        lax.fori_loop(0, ngen, gen_body, 0)
        plsc.subcore_barrier()
        pltpu.sync_copy(acc_sh.at[pl.ds(row0, _ROWS_PER_TILE)],
                        out_hbm.at[cid, pl.ds(row0, _ROWS_PER_TILE)])

    return _sc_scatter


def kernel(x, edge_index, edge_type, weight1, w_comp1, self_loop1,
           weight2, w_comp2, self_loop2):
    src = edge_index[0]
    dst = edge_index[1]
    pad = _E_PAD - _E
    gidx = jnp.concatenate(
        [edge_type * _N + src, jnp.zeros((pad,), jnp.int32)])
    sidx = jnp.concatenate(
        [dst, _N + (jnp.arange(pad, dtype=jnp.int32) % (_ACC_N - _N))])
    gidx = gidx.reshape(_NW * _NCH, _CHUNK)
    sidx = sidx.reshape(_NW * _NCH, _CHUNK)

    sc_scatter = _make_sc_scatter()
    y1 = _mm1(x, weight1, w_comp1, self_loop1)
    parts1 = sc_scatter(y1.reshape(_NG * _N, _D), gidx, sidx)
    h1, y2 = _mm2(parts1, parts1, y1, weight2, w_comp2, self_loop2)
    parts2 = sc_scatter(y2.reshape(_NG * _N, _D), gidx, sidx)
    h2 = _relu(parts2, parts2, y2)
    lane = jnp.arange(2, dtype=jnp.int32).reshape(1, 2, 1)
    return jnp.where(lane == 0, h1[:, None, :], h2[:, None, :])


# speedup vs baseline: 1.3171x; 1.0808x over previous
import functools

import jax
import jax.numpy as jnp
from jax import lax
from jax.experimental import pallas as pl
from jax.experimental.pallas import tpu as pltpu
from jax.experimental.pallas import tpu_sc as plsc

_N = 10000
_D = 128
_E = 160000
_NB = 4
_NG = 9
_ACC_N = 10240
_NW = 32
_CHUNK = 128
_E_PAD = 163840
_E_TILE = _E_PAD // _NW
_NCH = _E_TILE // _CHUNK
_ROWS_PER_TILE = _ACC_N // 16
_BN = 1000


def _project(x, w_ref, c_ref, s_ref, y_ref):
    xb = [jnp.dot(x, w_ref[b], preferred_element_type=jnp.float32)
          for b in range(_NB)]
    for r in range(_NG - 1):
        acc = c_ref[r, 0] * xb[0]
        for b in range(1, _NB):
            acc = acc + c_ref[r, b] * xb[b]
        y_ref[r, :, :] = acc
    y_ref[_NG - 1, :, :] = jnp.dot(
        x, s_ref[...], preferred_element_type=jnp.float32)


def _mm1_kernel(x_ref, w_ref, c_ref, s_ref, y_ref):
    _project(x_ref[...], w_ref, c_ref, s_ref, y_ref)


def _mm2_kernel(p0_ref, p1_ref, sp_ref, w_ref, c_ref, s_ref, h_ref, y_ref):
    h = jnp.maximum(p0_ref[0] + p1_ref[0] + sp_ref[0], 0.0)
    h_ref[...] = h
    _project(h, w_ref, c_ref, s_ref, y_ref)


def _relu_kernel(p0_ref, p1_ref, sp_ref, h_ref):
    h_ref[...] = jnp.maximum(p0_ref[0] + p1_ref[0] + sp_ref[0], 0.0)


_GRID = _N // _BN

_W_SPEC = pl.BlockSpec((_NB, _D, _D), lambda i: (0, 0, 0))
_C_SPEC = pl.BlockSpec(memory_space=pltpu.SMEM)
_S_SPEC = pl.BlockSpec((_D, _D), lambda i: (0, 0))
_X_SPEC = pl.BlockSpec((_BN, _D), lambda i: (i, 0))
_Y_SPEC = pl.BlockSpec((_NG, _BN, _D), lambda i: (0, i, 0))
_P0_SPEC = pl.BlockSpec((1, _BN, _D), lambda i: (0, i, 0))
_P1_SPEC = pl.BlockSpec((1, _BN, _D), lambda i: (1, i, 0))
_SP_SPEC = pl.BlockSpec((1, _BN, _D), lambda i: (_NG - 1, i, 0))

_mm1 = pl.pallas_call(
    _mm1_kernel,
    grid=(_GRID,),
    in_specs=[_X_SPEC, _W_SPEC, _C_SPEC, _S_SPEC],
    out_specs=_Y_SPEC,
    out_shape=jax.ShapeDtypeStruct((_NG, _N, _D), jnp.float32),
)

_mm2 = pl.pallas_call(
    _mm2_kernel,
    grid=(_GRID,),
    in_specs=[_P0_SPEC, _P1_SPEC, _SP_SPEC, _W_SPEC, _C_SPEC, _S_SPEC],
    out_specs=[_X_SPEC, _Y_SPEC],
    out_shape=[
        jax.ShapeDtypeStruct((_N, _D), jnp.float32),
        jax.ShapeDtypeStruct((_NG, _N, _D), jnp.float32),
    ],
)

_relu = pl.pallas_call(
    _relu_kernel,
    grid=(_GRID,),
    in_specs=[_P0_SPEC, _P1_SPEC, _SP_SPEC],
    out_specs=_X_SPEC,
    out_shape=jax.ShapeDtypeStruct((_N, _D), jnp.float32),
)


_DEPTH = 2
_Q0 = 64
_Q1 = (_NW * _NCH - 16 * _Q0) // 16
_QMAX = max(_Q0, _Q1)


@functools.cache
def _make_sc_scatter():
    mesh = plsc.VectorSubcoreMesh(core_axis_name="c", subcore_axis_name="s")

    @functools.partial(
        pl.kernel,
        out_type=jax.ShapeDtypeStruct((2, _ACC_N, _D), jnp.float32),
        mesh=mesh,
        scratch_types=(
            [pltpu.VMEM((_QMAX, _CHUNK), jnp.int32)] * 2
            + [pltpu.VMEM((_CHUNK, _D), jnp.float32)] * _DEPTH
            + [pltpu.VMEM_SHARED((_ACC_N, _D), jnp.float32)]
            + [pltpu.SemaphoreType.DMA] * (2 * _DEPTH)
        ),
    )
    def _sc_scatter(table_hbm, gidx_hbm, sidx_hbm, out_hbm,
                    gi_v, si_v, *scr):
        rows = scr[:_DEPTH]
        acc_sh = scr[_DEPTH]
        gsem = scr[_DEPTH + 1:2 * _DEPTH + 1]
        ssem = scr[2 * _DEPTH + 1:]
        cid = lax.axis_index("c")
        sid = lax.axis_index("s")
        ngen = jnp.where(cid == 0, _Q0 // _DEPTH, _Q1 // _DEPTH)

        def wait_gather(b):
            pltpu.make_async_copy(
                table_hbm.at[gi_v.at[b]], rows[b], gsem[b]).wait()

        def wait_scatter(b):
            pltpu.make_async_copy(
                rows[b], acc_sh.at[si_v.at[b]], ssem[b]).wait()

        @pl.when(cid == 0)
        def _():
            cbase = sid * _Q0
            pltpu.sync_copy(gidx_hbm.at[pl.ds(cbase, _Q0)],
                            gi_v.at[pl.ds(0, _Q0)])
            pltpu.sync_copy(sidx_hbm.at[pl.ds(cbase, _Q0)],
                            si_v.at[pl.ds(0, _Q0)])

        @pl.when(cid == 1)
        def _():
            cbase = 16 * _Q0 + sid * _Q1
            pltpu.sync_copy(gidx_hbm.at[pl.ds(cbase, _Q1)],
                            gi_v.at[pl.ds(0, _Q1)])
            pltpu.sync_copy(sidx_hbm.at[pl.ds(cbase, _Q1)],
                            si_v.at[pl.ds(0, _Q1)])

        zv = jnp.zeros((16,), jnp.float32)

        def zrow(i, carry):
            for j in range(8):
                rows[0][i, 16 * j:16 * (j + 1)] = zv
            return carry

        lax.fori_loop(0, _CHUNK, zrow, 0)
        row0 = pl.multiple_of(sid * _ROWS_PER_TILE, _ROWS_PER_TILE)
        for z in range(_ROWS_PER_TILE // _CHUNK):
            pltpu.async_copy(
                rows[0], acc_sh.at[pl.ds(row0 + z * _CHUNK, _CHUNK)],
                ssem[0])
        for z in range(_ROWS_PER_TILE // _CHUNK):
            pltpu.make_async_copy(
                rows[0], acc_sh.at[pl.ds(row0, _CHUNK)], ssem[0]).wait()
        for b in range(_DEPTH):
            pltpu.async_copy(table_hbm.at[gi_v.at[b]], rows[b], gsem[b])
        plsc.subcore_barrier()

        def gen_body(g, carry):
            base_j = g * _DEPTH
            for b in range(_DEPTH):
                wait_gather(b)
                pltpu.async_copy(
                    rows[b], acc_sh.at[si_v.at[base_j + b]], ssem[b],
                    add=True)
            for b in range(_DEPTH):
                wait_scatter(b)

                @pl.when(g + 1 < ngen)
                def _():
                    jn = base_j + _DEPTH + b
                    pltpu.async_copy(
                        table_hbm.at[gi_v.at[jn]], rows[b], gsem[b])
            return carry

        lax.fori_loop(0, ngen, gen_body, 0)
        plsc.subcore_barrier()
        pltpu.sync_copy(acc_sh.at[pl.ds(row0, _ROWS_PER_TILE)],
                        out_hbm.at[cid, pl.ds(row0, _ROWS_PER_TILE)])

    return _sc_scatter


def kernel(x, edge_index, edge_type, weight1, w_comp1, self_loop1,
           weight2, w_comp2, self_loop2):
    src = edge_index[0]
    dst = edge_index[1]
    pad = _E_PAD - _E
    gidx = jnp.concatenate(
        [edge_type * _N + src, jnp.zeros((pad,), jnp.int32)])
    sidx = jnp.concatenate(
        [dst, _N + (jnp.arange(pad, dtype=jnp.int32) % (_ACC_N - _N))])
    gidx = gidx.reshape(_NW * _NCH, _CHUNK)
    sidx = sidx.reshape(_NW * _NCH, _CHUNK)

    sc_scatter = _make_sc_scatter()
    y1 = _mm1(x, weight1, w_comp1, self_loop1)
    parts1 = sc_scatter(y1.reshape(_NG * _N, _D), gidx, sidx)
    h1, y2 = _mm2(parts1, parts1, y1, weight2, w_comp2, self_loop2)
    parts2 = sc_scatter(y2.reshape(_NG * _N, _D), gidx, sidx)
    h2 = _relu(parts2, parts2, y2)
    lane = jnp.arange(2, dtype=jnp.int32).reshape(1, 2, 1)
    return jnp.where(lane == 0, h1[:, None, :], h2[:, None, :])
